# baseline (speedup 1.0000x reference)
import jax
import jax.numpy as jnp
from jax import lax
from jax.experimental import pallas as pl
from jax.experimental.pallas import tpu as pltpu
from jax.experimental.pallas import tpu_sc as plsc

N = 10000
E = 320000
D = 128
H = 128
O = 128

NC = 2
NS = 16
NW = NC * NS
EPW = E // NW
LANES = 16

NB = 2000
NCHUNK = N // NB


UNROLL = 5


def _hist_body(idx_hbm, out_hbm, idx_v, local_v):
    cid = lax.axis_index("c")
    sid = lax.axis_index("s")
    wid = sid * NC + cid

    pltpu.sync_copy(idx_hbm.at[pl.ds(wid * EPW, EPW)], idx_v)

    zero16 = jnp.zeros((LANES,), jnp.float32)

    @plsc.parallel_loop(0, N // LANES, unroll=UNROLL)
    def _zero(i):
        local_v[0, pl.ds(i * LANES, LANES)] = zero16

    ones16 = jnp.ones((LANES,), jnp.float32)
    zi16 = jnp.zeros((LANES,), jnp.int32)

    @plsc.parallel_loop(0, EPW // LANES, unroll=UNROLL)
    def _hist(i):
        idx16 = idx_v[pl.ds(i * LANES, LANES)]
        plsc.addupdate_scatter(local_v, [zi16, idx16], ones16)

    pltpu.sync_copy(local_v, out_hbm.at[wid])


def _histogram(neighbor_dict):
    mesh = plsc.VectorSubcoreMesh(core_axis_name="c", subcore_axis_name="s")
    counts = pl.kernel(
        _hist_body,
        mesh=mesh,
        out_type=jax.ShapeDtypeStruct((NW, 1, N), jnp.float32),
        scratch_types=[
            pltpu.VMEM((EPW,), jnp.int32),
            pltpu.VMEM((1, N), jnp.float32),
        ],
        compiler_params=pltpu.CompilerParams(needs_layout_passes=False),
    )(neighbor_dict)
    return counts


def _z_body(feat_ref, w1_ref, b1_ref, w2_ref, b2_ref, z_ref):
    x = feat_ref[...]
    h = lax.dot_general(x, w1_ref[...], (((1,), (1,)), ((), ())),
                        preferred_element_type=jnp.float32,
                        precision=lax.Precision.DEFAULT) + b1_ref[...]
    h2 = lax.dot_general(h, w2_ref[...], (((1,), (1,)), ((), ())),
                         preferred_element_type=jnp.float32,
                         precision=lax.Precision.DEFAULT) + b2_ref[...]
    z_ref[...] = jnp.maximum(h2, 0.0).astype(jnp.bfloat16)


def _w_body(counts_ref, z_ref, wc_ref, bc_ref, out_ref):
    c = jnp.reshape(counts_ref[...], (NW, N))
    z = z_ref[...].astype(jnp.float32)
    acc = lax.dot_general(c, z, (((1,), (0,)), ((), ())),
                          preferred_element_type=jnp.float32,
                          precision=lax.Precision.DEFAULT)
    pooled = jnp.sum(acc, axis=0, keepdims=True) * (1.0 / E)
    fa = jnp.maximum(pooled, 0.0)
    out_ref[...] = lax.dot_general(fa, wc_ref[...], (((1,), (1,)), ((), ())),
                                   preferred_element_type=jnp.float32,
                                   precision=lax.Precision.HIGHEST) + bc_ref[...]


def kernel(feat, neighbor_dict, all_feat, W1, b1, W2, b2, Wc, bc):
    counts = _histogram(neighbor_dict)
    z = pl.pallas_call(
        _z_body,
        grid=(NCHUNK,),
        in_specs=[
            pl.BlockSpec((NB, D), lambda i: (i, 0)),
            pl.BlockSpec((H, D), lambda i: (0, 0)),
            pl.BlockSpec((1, H), lambda i: (0, 0)),
            pl.BlockSpec((H, H), lambda i: (0, 0)),
            pl.BlockSpec((1, H), lambda i: (0, 0)),
        ],
        out_specs=pl.BlockSpec((NB, H), lambda i: (i, 0)),
        out_shape=jax.ShapeDtypeStruct((N, H), jnp.bfloat16),
    )(all_feat, W1, b1.reshape(1, H), W2, b2.reshape(1, H))
    out = pl.pallas_call(
        _w_body,
        out_shape=jax.ShapeDtypeStruct((1, O), jnp.float32),
    )(counts, z, Wc, bc.reshape(1, O))
    return out.reshape(O)


# speedup vs baseline: 1.0007x; 1.0007x over previous
import jax
import jax.numpy as jnp
from jax import lax
from jax.experimental import pallas as pl
from jax.experimental.pallas import tpu as pltpu
from jax.experimental.pallas import tpu_sc as plsc

N = 10000
E = 320000
D = 128
H = 128
O = 128

NC = 2
NS = 16
NW = NC * NS
EPW = E // NW
LANES = 16

NB = 2000
NCHUNK = N // NB


UNROLL = 25


def _hist_body(idx_hbm, out_hbm, idx_v, local_v):
    cid = lax.axis_index("c")
    sid = lax.axis_index("s")
    wid = sid * NC + cid

    pltpu.sync_copy(idx_hbm.at[pl.ds(wid * EPW, EPW)], idx_v)

    zero16 = jnp.zeros((LANES,), jnp.float32)

    @plsc.parallel_loop(0, N // LANES, unroll=UNROLL)
    def _zero(i):
        local_v[0, pl.ds(i * LANES, LANES)] = zero16

    ones16 = jnp.ones((LANES,), jnp.float32)
    zi16 = jnp.zeros((LANES,), jnp.int32)

    @plsc.parallel_loop(0, EPW // LANES, unroll=UNROLL)
    def _hist(i):
        idx16 = idx_v[pl.ds(i * LANES, LANES)]
        plsc.addupdate_scatter(local_v, [zi16, idx16], ones16)

    pltpu.sync_copy(local_v, out_hbm.at[wid])


def _histogram(neighbor_dict):
    mesh = plsc.VectorSubcoreMesh(core_axis_name="c", subcore_axis_name="s")
    counts = pl.kernel(
        _hist_body,
        mesh=mesh,
        out_type=jax.ShapeDtypeStruct((NW, 1, N), jnp.float32),
        scratch_types=[
            pltpu.VMEM((EPW,), jnp.int32),
            pltpu.VMEM((1, N), jnp.float32),
        ],
        compiler_params=pltpu.CompilerParams(needs_layout_passes=False),
    )(neighbor_dict)
    return counts


def _z_body(feat_ref, w1_ref, b1_ref, w2_ref, b2_ref, z_ref):
    x = feat_ref[...]
    h = lax.dot_general(x, w1_ref[...], (((1,), (1,)), ((), ())),
                        preferred_element_type=jnp.float32,
                        precision=lax.Precision.DEFAULT) + b1_ref[...]
    h2 = lax.dot_general(h, w2_ref[...], (((1,), (1,)), ((), ())),
                         preferred_element_type=jnp.float32,
                         precision=lax.Precision.DEFAULT) + b2_ref[...]
    z_ref[...] = jnp.maximum(h2, 0.0).astype(jnp.bfloat16)


def _w_body(counts_ref, z_ref, wc_ref, bc_ref, out_ref):
    c = jnp.reshape(counts_ref[...], (NW, N))
    z = z_ref[...].astype(jnp.float32)
    acc = lax.dot_general(c, z, (((1,), (0,)), ((), ())),
                          preferred_element_type=jnp.float32,
                          precision=lax.Precision.DEFAULT)
    pooled = jnp.sum(acc, axis=0, keepdims=True) * (1.0 / E)
    fa = jnp.maximum(pooled, 0.0)
    out_ref[...] = lax.dot_general(fa, wc_ref[...], (((1,), (1,)), ((), ())),
                                   preferred_element_type=jnp.float32,
                                   precision=lax.Precision.HIGHEST) + bc_ref[...]


def kernel(feat, neighbor_dict, all_feat, W1, b1, W2, b2, Wc, bc):
    counts = _histogram(neighbor_dict)
    z = pl.pallas_call(
        _z_body,
        grid=(NCHUNK,),
        in_specs=[
            pl.BlockSpec((NB, D), lambda i: (i, 0)),
            pl.BlockSpec((H, D), lambda i: (0, 0)),
            pl.BlockSpec((1, H), lambda i: (0, 0)),
            pl.BlockSpec((H, H), lambda i: (0, 0)),
            pl.BlockSpec((1, H), lambda i: (0, 0)),
        ],
        out_specs=pl.BlockSpec((NB, H), lambda i: (i, 0)),
        out_shape=jax.ShapeDtypeStruct((N, H), jnp.bfloat16),
    )(all_feat, W1, b1.reshape(1, H), W2, b2.reshape(1, H))
    out = pl.pallas_call(
        _w_body,
        out_shape=jax.ShapeDtypeStruct((1, O), jnp.float32),
    )(counts, z, Wc, bc.reshape(1, O))
    return out.reshape(O)


# speedup vs baseline: 1.0029x; 1.0023x over previous
import jax
import jax.numpy as jnp
from jax import lax
from jax.experimental import pallas as pl
from jax.experimental.pallas import tpu as pltpu
from jax.experimental.pallas import tpu_sc as plsc

N = 10000
E = 320000
D = 128
H = 128
O = 128

NC = 2
NS = 16
NW = NC * NS
EPW = E // NW
LANES = 16

NB = 2000
NCHUNK = N // NB


UNROLL = 5


def _hist_body(idx_hbm, out_hbm, idx_v, local_v):
    cid = lax.axis_index("c")
    sid = lax.axis_index("s")
    wid = sid * NC + cid

    pltpu.sync_copy(idx_hbm.at[pl.ds(wid * EPW, EPW)], idx_v)

    zero16 = jnp.zeros((LANES,), jnp.float32)

    @plsc.parallel_loop(0, N // LANES, unroll=UNROLL)
    def _zero(i):
        local_v[0, pl.ds(i * LANES, LANES)] = zero16

    ones16 = jnp.ones((LANES,), jnp.float32)
    zi16 = jnp.zeros((LANES,), jnp.int32)

    @plsc.parallel_loop(0, EPW // LANES, unroll=UNROLL)
    def _hist(i):
        idx16 = idx_v[pl.ds(i * LANES, LANES)]
        plsc.addupdate_scatter(local_v, [zi16, idx16], ones16)

    pltpu.sync_copy(local_v, out_hbm.at[wid])


def _histogram(neighbor_dict):
    mesh = plsc.VectorSubcoreMesh(core_axis_name="c", subcore_axis_name="s")
    counts = pl.kernel(
        _hist_body,
        mesh=mesh,
        out_type=jax.ShapeDtypeStruct((NW, 1, N), jnp.float32),
        scratch_types=[
            pltpu.VMEM((EPW,), jnp.int32),
            pltpu.VMEM((1, N), jnp.float32),
        ],
        compiler_params=pltpu.CompilerParams(needs_layout_passes=False),
    )(neighbor_dict)
    return counts


def _z_body(feat_ref, w1_ref, b1_ref, w2_ref, b2_ref, z_ref):
    x = feat_ref[...]
    h = lax.dot_general(x, w1_ref[...], (((1,), (1,)), ((), ())),
                        preferred_element_type=jnp.float32,
                        precision=lax.Precision.DEFAULT) + b1_ref[...]
    h2 = lax.dot_general(h, w2_ref[...], (((1,), (1,)), ((), ())),
                         preferred_element_type=jnp.float32,
                         precision=lax.Precision.DEFAULT) + b2_ref[...]
    z_ref[...] = jnp.maximum(h2, 0.0).astype(jnp.bfloat16)


def _w_body(counts_ref, z_ref, wc_ref, bc_ref, out_ref):
    c = jnp.reshape(counts_ref[...], (NW, N))
    z = z_ref[...].astype(jnp.float32)
    acc = lax.dot_general(c, z, (((1,), (0,)), ((), ())),
                          preferred_element_type=jnp.float32,
                          precision=lax.Precision.DEFAULT)
    pooled = jnp.sum(acc, axis=0, keepdims=True) * (1.0 / E)
    fa = jnp.maximum(pooled, 0.0)
    out_ref[...] = lax.dot_general(fa, wc_ref[...], (((1,), (1,)), ((), ())),
                                   preferred_element_type=jnp.float32,
                                   precision=lax.Precision.HIGHEST) + bc_ref[...]


def kernel(feat, neighbor_dict, all_feat, W1, b1, W2, b2, Wc, bc):
    counts = _histogram(neighbor_dict)
    z = pl.pallas_call(
        _z_body,
        grid=(NCHUNK,),
        in_specs=[
            pl.BlockSpec((NB, D), lambda i: (i, 0)),
            pl.BlockSpec((H, D), lambda i: (0, 0)),
            pl.BlockSpec((1, H), lambda i: (0, 0)),
            pl.BlockSpec((H, H), lambda i: (0, 0)),
            pl.BlockSpec((1, H), lambda i: (0, 0)),
        ],
        out_specs=pl.BlockSpec((NB, H), lambda i: (i, 0)),
        out_shape=jax.ShapeDtypeStruct((N, H), jnp.bfloat16),
    )(all_feat, W1, b1.reshape(1, H), W2, b2.reshape(1, H))
    out = pl.pallas_call(
        _w_body,
        out_shape=jax.ShapeDtypeStruct((1, O), jnp.float32),
    )(counts, z, Wc, bc.reshape(1, O))
    return out.reshape(O)
